# baseline (speedup 1.0000x reference)
import functools

import jax
import jax.numpy as jnp
from jax import lax
from jax.experimental import pallas as pl
from jax.experimental.pallas import tpu as pltpu
from jax.experimental.pallas import tpu_sc as plsc

BATCH = 16384
DIM = 32
NSLOT = 8
NC, NS = 2, 16
NW = NC * NS
BPW = BATCH // NW
CHUNK = 64
NCHUNK = BPW // CHUNK
OUT_D = NSLOT * DIM
NBUF = 4

_mesh = plsc.VectorSubcoreMesh(core_axis_name="c", subcore_axis_name="s")

PACK = 4
WIDE = PACK * DIM


def _pack4_body(in_ref, out_ref, sub):
    blk = in_ref.shape[0]
    row = lax.broadcasted_iota(jnp.int32, (sub // PACK, sub), 0)
    col = lax.broadcasted_iota(jnp.int32, (sub // PACK, sub), 1)
    sel = [(col == PACK * row + a).astype(jnp.float32) for a in range(PACK)]
    for t in range(blk // sub):
        x = in_ref[pl.ds(t * sub, sub), :]
        parts = [
            jax.lax.dot(sel[a], x, precision=jax.lax.Precision.HIGHEST)
            for a in range(PACK)
        ]
        out_ref[pl.ds(t * (sub // PACK), sub // PACK), :] = (
            jnp.concatenate(parts, axis=1))


def _linearize(table, blk=8192):
    v = table.shape[0]
    if v % blk != 0:
        blk = 8000 if v % 8000 == 0 else v
    sub = 256 if blk % 256 == 0 else 1000
    packed = pl.pallas_call(
        functools.partial(_pack4_body, sub=sub),
        grid=(v // blk,),
        in_specs=[pl.BlockSpec((blk, DIM), lambda i: (i, 0))],
        out_specs=pl.BlockSpec((blk // PACK, WIDE), lambda i: (i, 0)),
        out_shape=jax.ShapeDtypeStruct((v // PACK, WIDE), jnp.float32),
    )(table)
    return packed.reshape(v, DIM)


@functools.partial(
    pl.kernel,
    out_type=jax.ShapeDtypeStruct((BATCH, OUT_D), jnp.float32),
    mesh=_mesh,
    scratch_types=[
        pltpu.VMEM((NSLOT * BPW,), jnp.int32),
        pltpu.VMEM((NBUF, NSLOT, CHUNK, DIM), jnp.float32),
        pltpu.SemaphoreType.DMA,
        pltpu.SemaphoreType.DMA,
        pltpu.SemaphoreType.DMA,
    ],
    compiler_params=pltpu.CompilerParams(use_tc_tiling_on_sc=False),
)
def _gather_concat(idx_hbm, paper_hbm, pfield_hbm, author_hbm, year_hbm,
                   oa_hbm, out_hbm, idx_v, slot_v, gsem, wsem0, wsem1):
    wid = lax.axis_index("s") * NC + lax.axis_index("c")
    base = wid * BPW
    tables = (paper_hbm, pfield_hbm, pfield_hbm, author_hbm, author_hbm,
              author_hbm, year_hbm, oa_hbm)
    wsems = (wsem0, wsem1)
    pltpu.sync_copy(idx_hbm.at[pl.ds(wid * NSLOT * BPW, NSLOT * BPW)], idx_v)

    def issue_gathers(c):
        buf = c % NBUF
        return [
            pltpu.async_copy(
                tab.at[idx_v.at[pl.ds(s * BPW + c * CHUNK, CHUNK)]],
                slot_v.at[buf, s], gsem)
            for s, tab in enumerate(tables)
        ]

    def issue_writes(c):
        buf = c % NBUF
        rb = base + c * CHUNK
        return [
            pltpu.async_copy(
                slot_v.at[buf, s],
                out_hbm.at[pl.ds(rb, CHUNK), pl.ds(s * DIM, DIM)],
                wsems[c % 2])
            for s in range(NSLOT)
        ]

    gathers = [None] * NCHUNK
    writes = [None] * NCHUNK
    for b in range(NBUF - 1):
        gathers[b] = issue_gathers(b)
    for c in range(NCHUNK):
        n = c + NBUF - 1
        if n < NCHUNK:
            if c >= 1 and writes[c - 1] is not None:
                for w in writes[c - 1]:
                    w.wait()
            gathers[n] = issue_gathers(n)
        for g in gathers[c]:
            g.wait()
        writes[c] = issue_writes(c)
    for c in range(NCHUNK):
        if writes[c] is not None and c >= NCHUNK - NBUF:
            for w in writes[c]:
                w.wait()


def kernel(paperId, fieldsOfStudy_0, fieldsOfStudy_1, authors_0, authors_1,
           authors_2, year, isOpenAccess, paper_table, pfield_table,
           author_table, year_table, oa_table):
    idx = jnp.stack([paperId, fieldsOfStudy_0, fieldsOfStudy_1, authors_0,
                     authors_1, authors_2, year, isOpenAccess])
    idx = (idx.astype(jnp.int32)
              .reshape(NSLOT, NW, BPW)
              .transpose(1, 0, 2)
              .reshape(-1))
    def pad_rows(t):
        v = t.shape[0]
        vp = -(-v // 256) * 256
        return jnp.pad(t, ((0, vp - v), (0, 0))) if vp != v else t

    return _gather_concat(idx, _linearize(paper_table),
                          _linearize(pad_rows(pfield_table)),
                          _linearize(author_table),
                          _linearize(pad_rows(year_table)),
                          _linearize(pad_rows(oa_table)))


# speedup vs baseline: 6.6916x; 6.6916x over previous
import functools

import jax
import jax.numpy as jnp
from jax import lax
from jax.experimental import pallas as pl
from jax.experimental.pallas import tpu as pltpu
from jax.experimental.pallas import tpu_sc as plsc

BATCH = 16384
DIM = 32
NSLOT = 8
NC, NS = 2, 16
NW = NC * NS
BPW = BATCH // NW
CHUNK = 64
NCHUNK = BPW // CHUNK
OUT_D = NSLOT * DIM
NBUF = 4

_mesh = plsc.VectorSubcoreMesh(core_axis_name="c", subcore_axis_name="s")

PACK = 4
WIDE = PACK * DIM


def _linearize(table, one):
    v = table.shape[0]
    return (table.reshape(v // PACK, WIDE) * one).reshape(v, DIM)


@functools.partial(
    pl.kernel,
    out_type=jax.ShapeDtypeStruct((BATCH, OUT_D), jnp.float32),
    mesh=_mesh,
    scratch_types=[
        pltpu.VMEM((NSLOT * BPW,), jnp.int32),
        pltpu.VMEM((NBUF, NSLOT, CHUNK, DIM), jnp.float32),
        pltpu.SemaphoreType.DMA,
        pltpu.SemaphoreType.DMA,
        pltpu.SemaphoreType.DMA,
    ],
    compiler_params=pltpu.CompilerParams(use_tc_tiling_on_sc=False),
)
def _gather_concat(idx_hbm, paper_hbm, pfield_hbm, author_hbm, year_hbm,
                   oa_hbm, out_hbm, idx_v, slot_v, gsem, wsem0, wsem1):
    wid = lax.axis_index("s") * NC + lax.axis_index("c")
    base = wid * BPW
    tables = (paper_hbm, pfield_hbm, pfield_hbm, author_hbm, author_hbm,
              author_hbm, year_hbm, oa_hbm)
    wsems = (wsem0, wsem1)
    pltpu.sync_copy(idx_hbm.at[pl.ds(wid * NSLOT * BPW, NSLOT * BPW)], idx_v)

    def issue_gathers(c):
        buf = c % NBUF
        return [
            pltpu.async_copy(
                tab.at[idx_v.at[pl.ds(s * BPW + c * CHUNK, CHUNK)]],
                slot_v.at[buf, s], gsem)
            for s, tab in enumerate(tables)
        ]

    def issue_writes(c):
        buf = c % NBUF
        rb = base + c * CHUNK
        return [
            pltpu.async_copy(
                slot_v.at[buf, s],
                out_hbm.at[pl.ds(rb, CHUNK), pl.ds(s * DIM, DIM)],
                wsems[c % 2])
            for s in range(NSLOT)
        ]

    gathers = [None] * NCHUNK
    writes = [None] * NCHUNK
    for b in range(NBUF - 1):
        gathers[b] = issue_gathers(b)
    for c in range(NCHUNK):
        n = c + NBUF - 1
        if n < NCHUNK:
            if c >= 1 and writes[c - 1] is not None:
                for w in writes[c - 1]:
                    w.wait()
            gathers[n] = issue_gathers(n)
        for g in gathers[c]:
            g.wait()
        writes[c] = issue_writes(c)
    for c in range(NCHUNK):
        if writes[c] is not None and c >= NCHUNK - NBUF:
            for w in writes[c]:
                w.wait()


def kernel(paperId, fieldsOfStudy_0, fieldsOfStudy_1, authors_0, authors_1,
           authors_2, year, isOpenAccess, paper_table, pfield_table,
           author_table, year_table, oa_table):
    idx = jnp.stack([paperId, fieldsOfStudy_0, fieldsOfStudy_1, authors_0,
                     authors_1, authors_2, year, isOpenAccess])
    idx = (idx.astype(jnp.int32)
              .reshape(NSLOT, NW, BPW)
              .transpose(1, 0, 2)
              .reshape(-1))
    one = (paperId[0] * 0 + 1).astype(jnp.float32)
    oa_pad = jnp.pad(oa_table, ((0, 1), (0, 0)))
    return _gather_concat(idx, _linearize(paper_table, one),
                          _linearize(pfield_table, one),
                          _linearize(author_table, one),
                          _linearize(year_table, one),
                          _linearize(oa_pad, one))
